# baseline (speedup 1.0000x reference)
import functools
import math

import jax
import jax.numpy as jnp
from jax import lax
from jax.experimental import pallas as pl
from jax.experimental.pallas import tpu as pltpu
from jax.experimental.pallas import tpu_sc as plsc

PAD = 1
SEQ = 2048
EMB = 1024
NROW = 2050
NROW_PAD = 2176

SC_BATCHES = 2
NUM_WORKERS = 32
ROWS_PER_WORKER = SC_BATCHES * SEQ // NUM_WORKERS
CHUNK = 16
NCHUNK = ROWS_PER_WORKER // CHUNK
NBUF = 6
SPANS_PER_ROW = SEQ // ROWS_PER_WORKER


def _sc_body(ids_hbm, table_hbm, out_hbm, ids_v, idx_v, rows_v, gsem, osem):
    c = lax.axis_index("c")
    s = lax.axis_index("s")
    wid = s * 2 + c
    b = wid // SPANS_PER_ROW
    soff = wid % SPANS_PER_ROW
    row_base = b * SEQ

    pltpu.sync_copy(ids_hbm.at[pl.ds(row_base, SEQ)], ids_v)

    def pref_body(j, carry):
        v = ids_v[pl.ds(j * 16, 16)]
        mi = jnp.where(v != PAD, jnp.full((16,), 1, jnp.int32),
                       jnp.zeros((16,), jnp.int32))
        return carry + jnp.sum(mi)

    carry = lax.fori_loop(0, soff * (ROWS_PER_WORKER // 16), pref_body,
                          jnp.int32(0))

    span = soff * ROWS_PER_WORKER

    def span_body(j, carry):
        v = ids_v[pl.ds(span + j * 16, 16)]
        mi = jnp.where(v != PAD, jnp.full((16,), 1, jnp.int32),
                       jnp.zeros((16,), jnp.int32))
        cum = plsc.cumsum(mi)
        idx_v[pl.ds(j * 16, 16)] = (carry + cum) * mi + PAD
        return carry + jnp.sum(mi)

    lax.fori_loop(0, ROWS_PER_WORKER // 16, span_body, carry)

    out_base = wid * ROWS_PER_WORKER

    def fire_gather(k):
        return pltpu.async_copy(
            table_hbm.at[idx_v.at[pl.ds(k * CHUNK, CHUNK)]],
            rows_v.at[k % NBUF], gsem.at[k % NBUF])

    def fire_out(k):
        return pltpu.async_copy(
            rows_v.at[k % NBUF], out_hbm.at[pl.ds(out_base + k * CHUNK, CHUNK)],
            osem.at[k % NBUF])

    gathers = [None] * NCHUNK
    outs = [None] * NCHUNK
    for k in range(min(NBUF - 1, NCHUNK)):
        gathers[k] = fire_gather(k)
    for k in range(NCHUNK):
        if k >= 1:
            outs[k - 1].wait()
        nxt = k + NBUF - 1
        if nxt < NCHUNK:
            gathers[nxt] = fire_gather(nxt)
        gathers[k].wait()
        outs[k] = fire_out(k)
    outs[NCHUNK - 1].wait()


_sc_call = functools.partial(
    pl.kernel,
    out_type=jax.ShapeDtypeStruct((NUM_WORKERS * ROWS_PER_WORKER, EMB),
                                  jnp.float32),
    mesh=plsc.VectorSubcoreMesh(core_axis_name="c", subcore_axis_name="s"),
    compiler_params=pltpu.CompilerParams(needs_layout_passes=False),
    scratch_types=[
        pltpu.VMEM((SEQ,), jnp.int32),
        pltpu.VMEM((ROWS_PER_WORKER,), jnp.int32),
        pltpu.VMEM((NBUF, CHUNK, EMB), jnp.float32),
        pltpu.SemaphoreType.DMA((NBUF,)),
        pltpu.SemaphoreType.DMA((NBUF,)),
    ],
)(_sc_body)


def _tc_body(ids_ref, tri_ref, wbf_ref, out_ref):
    ids = ids_ref[0, :, :]
    mi = jnp.minimum(jnp.abs(ids - PAD), 1)
    mbf = mi.astype(jnp.bfloat16)
    cum = jnp.dot(tri_ref[...], mbf, preferred_element_type=jnp.float32)
    mf = mi.astype(jnp.float32)
    pos = cum * mf + jnp.float32(PAD)
    col = lax.broadcasted_iota(jnp.int32, (SEQ, NROW_PAD), 1).astype(
        jnp.float32)
    onehot = jnp.maximum(
        jnp.float32(1) - jnp.abs(pos - col), jnp.float32(0)
    ).astype(jnp.bfloat16)
    out_ref[0, :, :] = jnp.dot(onehot, wbf_ref[...],
                               preferred_element_type=jnp.float32)


def _tc_call(ids3d, tri, wbf):
    n = ids3d.shape[0]
    return pl.pallas_call(
        _tc_body,
        grid=(n,),
        in_specs=[
            pl.BlockSpec((1, SEQ, 1), lambda i: (i, 0, 0)),
            pl.BlockSpec((SEQ, SEQ), lambda i: (0, 0)),
            pl.BlockSpec((NROW_PAD, EMB), lambda i: (0, 0)),
        ],
        out_specs=pl.BlockSpec((1, SEQ, EMB), lambda i: (i, 0, 0)),
        out_shape=jax.ShapeDtypeStruct((n, SEQ, EMB), jnp.float32),
    )(ids3d, tri, wbf)


@jax.jit
def kernel(input_ids, weight):
    bsz, seq_len = input_ids.shape
    ids = input_ids.astype(jnp.int32)
    sc_out = _sc_call(ids[:SC_BATCHES].reshape(-1), weight)

    r = lax.broadcasted_iota(jnp.int32, (SEQ, SEQ), 0)
    c = lax.broadcasted_iota(jnp.int32, (SEQ, SEQ), 1)
    tri = (r >= c).astype(jnp.bfloat16)
    wbf = jnp.pad(weight, ((0, NROW_PAD - NROW), (0, 0))).astype(jnp.bfloat16)
    ids_tc = ids[SC_BATCHES:].reshape(bsz - SC_BATCHES, seq_len, 1)
    tc_out = _tc_call(ids_tc, tri, wbf)

    return jnp.concatenate(
        [sc_out.reshape(SC_BATCHES, seq_len, EMB), tc_out], axis=0)


# speedup vs baseline: 1.9799x; 1.9799x over previous
import functools

import jax
import jax.numpy as jnp
from jax import lax
from jax.experimental import pallas as pl
from jax.experimental.pallas import tpu as pltpu
from jax.experimental.pallas import tpu_sc as plsc

PAD = 1
NUM_WORKERS = 32
ROWS_PER_WORKER = 256
CHUNK = 16
NCHUNK = ROWS_PER_WORKER // CHUNK
NBUF = 7
SEQ = 2048
EMB = 1024
SPANS_PER_ROW = SEQ // ROWS_PER_WORKER


def _sc_body(ids_hbm, table_hbm, out_hbm, ids_v, idx_v, rows_v, gsem, osem):
    c = lax.axis_index("c")
    s = lax.axis_index("s")
    wid = s * 2 + c
    b = wid // SPANS_PER_ROW
    soff = wid % SPANS_PER_ROW
    row_base = b * SEQ

    pltpu.sync_copy(ids_hbm.at[pl.ds(row_base, SEQ)], ids_v)

    def pref_body(j, carry):
        v = ids_v[pl.ds(j * 16, 16)]
        mi = jnp.where(v != PAD, jnp.full((16,), 1, jnp.int32),
                       jnp.zeros((16,), jnp.int32))
        return carry + jnp.sum(mi)

    carry = lax.fori_loop(0, soff * (ROWS_PER_WORKER // 16), pref_body,
                          jnp.int32(0))

    span = soff * ROWS_PER_WORKER

    def span_body(j, carry):
        v = ids_v[pl.ds(span + j * 16, 16)]
        mi = jnp.where(v != PAD, jnp.full((16,), 1, jnp.int32),
                       jnp.zeros((16,), jnp.int32))
        cum = plsc.cumsum(mi)
        idx_v[pl.ds(j * 16, 16)] = (carry + cum) * mi + PAD
        return carry + jnp.sum(mi)

    lax.fori_loop(0, ROWS_PER_WORKER // 16, span_body, carry)

    out_base = wid * ROWS_PER_WORKER

    def fire_gather(k):
        return pltpu.async_copy(
            table_hbm.at[idx_v.at[pl.ds(k * CHUNK, CHUNK)]],
            rows_v.at[k % NBUF], gsem.at[k % NBUF])

    def fire_out(k):
        return pltpu.async_copy(
            rows_v.at[k % NBUF], out_hbm.at[pl.ds(out_base + k * CHUNK, CHUNK)],
            osem.at[k % NBUF])

    gathers = [None] * NCHUNK
    outs = [None] * NCHUNK
    for k in range(min(NBUF - 1, NCHUNK)):
        gathers[k] = fire_gather(k)
    for k in range(NCHUNK):
        if k >= 1:
            outs[k - 1].wait()
        nxt = k + NBUF - 1
        if nxt < NCHUNK:
            gathers[nxt] = fire_gather(nxt)
        gathers[k].wait()
        outs[k] = fire_out(k)
    outs[NCHUNK - 1].wait()


_sc_call = functools.partial(
    pl.kernel,
    out_type=jax.ShapeDtypeStruct((NUM_WORKERS * ROWS_PER_WORKER, EMB),
                                  jnp.float32),
    mesh=plsc.VectorSubcoreMesh(core_axis_name="c", subcore_axis_name="s"),
    compiler_params=pltpu.CompilerParams(needs_layout_passes=False),
    scratch_types=[
        pltpu.VMEM((SEQ,), jnp.int32),
        pltpu.VMEM((ROWS_PER_WORKER,), jnp.int32),
        pltpu.VMEM((NBUF, CHUNK, EMB), jnp.float32),
        pltpu.SemaphoreType.DMA((NBUF,)),
        pltpu.SemaphoreType.DMA((NBUF,)),
    ],
)(_sc_body)


@jax.jit
def kernel(input_ids, weight):
    bsz, seq_len = input_ids.shape
    ids = input_ids.reshape(-1).astype(jnp.int32)
    out = _sc_call(ids, weight)
    return out.reshape(bsz, seq_len, weight.shape[-1])
